# baseline (speedup 1.0000x reference)
import functools

import jax
import jax.numpy as jnp
from jax import lax
from jax.experimental import pallas as pl
from jax.experimental.pallas import tpu as pltpu
from jax.experimental.pallas import tpu_sc as plsc

_B = 1024
_NS1 = 10
_NS0 = 25
_D = 128
_NC, _NS = 2, 16
_NW = _NC * _NS
_NB = _B // _NW
_S1 = _NB * _NS1
_CH = 8
_ROWS = _CH * _NS0
_NCHUNK = _S1 // _CH

_INTERPRET = False


def _sc_body(feat, adjv, bat, h0_o, h1_o, s2_o,
             ids_v, h0r, adj1, s1, adj2, adj2b, s2,
             rb0, rb1, rb2, rb3, st0, st1,
             sem_a, sg0, sg1, sg2, sg3, ss0, ss1):
    wid = lax.axis_index("s") * _NC + lax.axis_index("c")
    base_b = pl.multiple_of(wid * _NB, 8)
    lane = lax.iota(jnp.int32, 16)
    c25 = jnp.full((16,), _NS0, jnp.int32)

    pltpu.sync_copy(bat.at[pl.ds(base_b, _NB)], ids_v)
    cp1 = pltpu.async_copy(adjv.at[ids_v], adj1, sem_a)
    cp2 = pltpu.async_copy(feat.at[ids_v], h0r, sem_a)
    cp1.wait()
    cp2.wait()
    pltpu.sync_copy(h0r, h0_o.at[pl.ds(base_b, _NB)])

    for g in range(_S1 // 16):
        k = lane + g * 16
        li = k & 31
        j = lax.shift_right_logical(k, 5)
        s1[pl.ds(g * 16, 16)] = plsc.load_gather(adj1, [li, j])

    cps = []
    for off, n, dst, doff in ((0, 128, rb0, 0), (128, 64, rb0, 128),
                              (192, 128, rb1, 0)):
        cps.append(pltpu.async_copy(feat.at[s1.at[pl.ds(off, n)]],
                                    dst.at[pl.ds(doff, n)], sem_a))

    adjs, asems = (adj2, adj2b), (sg0, sg1)
    acps = [pltpu.async_copy(adjv.at[s1.at[pl.ds(0, 80)]], adj2, sg0), None]

    for c in cps:
        c.wait()
    hcps = []
    for j in range(_NS1):
        srcbuf = rb0 if j < 6 else rb1
        soff = j * _NB if j < 6 else (j - 6) * _NB
        hcps.append(pltpu.async_copy(srcbuf.at[pl.ds(soff, _NB)],
                                     h1_o.at[pl.ds(j * _B + base_b, _NB)],
                                     sem_a))

    for q in range(4):
        if q + 1 < 4:
            acps[(q + 1) % 2] = pltpu.async_copy(
                adjv.at[s1.at[pl.ds((q + 1) * 80, 80)]],
                adjs[(q + 1) % 2], asems[(q + 1) % 2])
        acps[q % 2].wait()
        a2 = adjs[q % 2]

        def ext_body(g, _, q=q, a2=a2):
            k = lane + g * 16
            slot = lax.div(k, c25)
            j = k - slot * _NS0
            vals = plsc.load_gather(a2, [slot, j])
            s2[pl.ds(pl.multiple_of(q * 2000 + g * 16, 8), 16)] = vals
            return 0
        lax.fori_loop(0, 2000 // 16, ext_body, 0)

    for c in hcps:
        c.wait()

    rbs = (rb0, rb1, rb2, rb3)
    sts, sgs, sss = (st0, st1), (sg0, sg1, sg2, sg3), (ss0, ss1)

    def start_gather(c, b):
        for doff, n in ((0, 64), (64, 64), (128, 72)):
            off = pl.multiple_of(c * _ROWS + doff, 8)
            pltpu.async_copy(feat.at[s2.at[pl.ds(off, n)]],
                             rbs[b].at[pl.ds(doff, n)], sgs[b])

    def wait_gather(b):
        pltpu.make_async_copy(feat.at[pl.ds(0, _ROWS)], rbs[b], sgs[b]).wait()

    def wait_store(b):
        pltpu.make_async_copy(sts[b], s2_o.at[pl.ds(0, _CH)], sss[b]).wait()

    def acc_chunk(rb, st):
        def seg_body(sl, _):
            rbase = sl * _NS0

            def row5_body(r5, accs):
                row = rbase + r5 * 5
                for r in range(5):
                    accs = tuple(accs[v] + rb[row + r, pl.ds(v * 16, 16)]
                                 for v in range(8))
                return accs

            accs = lax.fori_loop(
                0, _NS0 // 5, row5_body,
                tuple(jnp.zeros((16,), jnp.float32) for _ in range(8)))
            for v in range(8):
                st[sl, pl.ds(v * 16, 16)] = accs[v]
            return 0
        lax.fori_loop(0, _CH, seg_body, 0)

    start_gather(0, 0)
    start_gather(1, 1)
    start_gather(2, 2)

    def quad_body(p, _):
        for b in range(4):
            c = p * 4 + b

            @pl.when(c + 3 < _NCHUNK)
            def _():
                start_gather(c + 3, (b + 3) % 4)

            wait_gather(b)

            @pl.when(c >= 2)
            def _():
                wait_store(b % 2)

            acc_chunk(rbs[b], sts[b % 2])
            row0 = (lax.shift_right_logical(c, 2) * _B + base_b
                    + lax.shift_left(c & 3, 3))
            pltpu.async_copy(
                sts[b % 2], s2_o.at[pl.ds(pl.multiple_of(row0, 8), _CH)],
                sss[b % 2])
        return 0

    lax.fori_loop(0, _NCHUNK // 4, quad_body, 0)
    wait_store(0)
    wait_store(1)


def _sc_stage(features, adj, batch1):
    f32, i32 = jnp.float32, jnp.int32
    fn = functools.partial(
        pl.kernel,
        out_type=[
            jax.ShapeDtypeStruct((_B, _D), f32),
            jax.ShapeDtypeStruct((_B * _NS1, _D), f32),
            jax.ShapeDtypeStruct((_B * _NS1, _D), f32),
        ],
        mesh=plsc.VectorSubcoreMesh(core_axis_name="c", subcore_axis_name="s"),
        compiler_params=pltpu.CompilerParams(needs_layout_passes=False, use_tc_tiling_on_sc=False),
        scratch_types=[
            pltpu.VMEM((_NB,), i32),
            pltpu.VMEM((_NB, _D), f32),
            pltpu.VMEM((_NB, 32), i32),
            pltpu.VMEM((_S1,), i32),
            pltpu.VMEM((80, 32), i32),
            pltpu.VMEM((80, 32), i32),
            pltpu.VMEM((_S1 * _NS0,), i32),
            pltpu.VMEM((_ROWS, _D), f32),
            pltpu.VMEM((_ROWS, _D), f32),
            pltpu.VMEM((_ROWS, _D), f32),
            pltpu.VMEM((_ROWS, _D), f32),
            pltpu.VMEM((_CH, _D), f32),
            pltpu.VMEM((_CH, _D), f32),
            pltpu.SemaphoreType.DMA,
            pltpu.SemaphoreType.DMA,
            pltpu.SemaphoreType.DMA,
            pltpu.SemaphoreType.DMA,
            pltpu.SemaphoreType.DMA,
            pltpu.SemaphoreType.DMA,
            pltpu.SemaphoreType.DMA,
        ],
    )(_sc_body)
    return fn(features, adj, batch1)


def _tc_body(h0_ref, h1_ref, s2sum_ref, ws0_ref, wn0_ref, ws1_ref, wn1_ref,
             out_ref):
    ws0 = ws0_ref[...]
    wn0 = wn0_ref[...]
    h0 = h0_ref[...]
    h1 = h1_ref[...]
    nm0 = h1[:_B]
    for j in range(1, _NS1):
        nm0 = nm0 + h1[j * _B:(j + 1) * _B]
    nm0 = nm0 * (1.0 / _NS1)
    a00 = jax.nn.relu(jnp.dot(h0, ws0, preferred_element_type=jnp.float32))
    b00 = jax.nn.relu(jnp.dot(nm0, wn0, preferred_element_type=jnp.float32))
    nm1 = s2sum_ref[...] * (1.0 / _NS0)
    a01 = jax.nn.relu(jnp.dot(h1, ws0, preferred_element_type=jnp.float32))
    b01 = jax.nn.relu(jnp.dot(nm1, wn0, preferred_element_type=jnp.float32))
    nma = a01[:_B]
    nmb = b01[:_B]
    for j in range(1, _NS1):
        nma = nma + a01[j * _B:(j + 1) * _B]
        nmb = nmb + b01[j * _B:(j + 1) * _B]
    nma = nma * (1.0 / _NS1)
    nmb = nmb * (1.0 / _NS1)
    ws1 = ws1_ref[...]
    wn1 = wn1_ref[...]
    fs = (jnp.dot(a00, ws1[:_D], preferred_element_type=jnp.float32)
          + jnp.dot(b00, ws1[_D:], preferred_element_type=jnp.float32))
    fn = (jnp.dot(nma, wn1[:_D], preferred_element_type=jnp.float32)
          + jnp.dot(nmb, wn1[_D:], preferred_element_type=jnp.float32))
    out = jnp.concatenate([fs, fn], axis=1)
    norm2 = jnp.sum(out * out, axis=1, keepdims=True)
    out_ref[...] = out * jax.lax.rsqrt(jnp.maximum(norm2, 1e-24))


def _dense_stage(h0, h1, s2sum, ws0, wn0, ws1, wn1):
    return pl.pallas_call(
        _tc_body,
        out_shape=jax.ShapeDtypeStruct((_B, 2 * _D), jnp.float32),
        interpret=_INTERPRET,
    )(h0, h1, s2sum, ws0, wn0, ws1, wn1)


def kernel(features, adj, batch1, W_self_0, W_neigh_0, W_self_1, W_neigh_1):
    h0, h1, s2sum = _sc_stage(features, adj, batch1)
    return _dense_stage(h0, h1, s2sum, W_self_0, W_neigh_0, W_self_1, W_neigh_1)


# speedup vs baseline: 1.0002x; 1.0002x over previous
import functools

import jax
import jax.numpy as jnp
from jax import lax
from jax.experimental import pallas as pl
from jax.experimental.pallas import tpu as pltpu
from jax.experimental.pallas import tpu_sc as plsc

_B = 1024
_NS1 = 10
_NS0 = 25
_D = 128
_NC, _NS = 2, 16
_NW = _NC * _NS
_NB = _B // _NW
_S1 = _NB * _NS1
_CH = 8
_ROWS = _CH * _NS0
_NCHUNK = _S1 // _CH


def _sc_body(feat, adjv, bat, h0_o, h1_o, s2_o,
             ids_v, h0r, adj1, s1, adj2, adj2b, s2,
             rb0, rb1, rb2, rb3, st0, st1,
             sem_a, sg0, sg1, sg2, sg3, ss0, ss1):
    wid = lax.axis_index("s") * _NC + lax.axis_index("c")
    base_b = pl.multiple_of(wid * _NB, 8)
    lane = lax.iota(jnp.int32, 16)
    c25 = jnp.full((16,), _NS0, jnp.int32)

    pltpu.sync_copy(bat.at[pl.ds(base_b, _NB)], ids_v)
    cp1 = pltpu.async_copy(adjv.at[ids_v], adj1, sem_a)
    cp2 = pltpu.async_copy(feat.at[ids_v], h0r, sem_a)
    cp1.wait()
    cp2.wait()
    pltpu.sync_copy(h0r, h0_o.at[pl.ds(base_b, _NB)])

    for g in range(_S1 // 16):
        k = lane + g * 16
        li = k & 31
        j = lax.shift_right_logical(k, 5)
        s1[pl.ds(g * 16, 16)] = plsc.load_gather(adj1, [li, j])

    cps = []
    for off, n, dst, doff in ((0, 128, rb0, 0), (128, 64, rb0, 128),
                              (192, 128, rb1, 0)):
        cps.append(pltpu.async_copy(feat.at[s1.at[pl.ds(off, n)]],
                                    dst.at[pl.ds(doff, n)], sem_a))

    adjs, asems = (adj2, adj2b), (sg0, sg1)
    acps = [pltpu.async_copy(adjv.at[s1.at[pl.ds(0, 80)]], adj2, sg0), None]

    for c in cps:
        c.wait()
    hcps = []
    for j in range(_NS1):
        srcbuf = rb0 if j < 6 else rb1
        soff = j * _NB if j < 6 else (j - 6) * _NB
        hcps.append(pltpu.async_copy(srcbuf.at[pl.ds(soff, _NB)],
                                     h1_o.at[pl.ds(j * _B + base_b, _NB)],
                                     sem_a))

    for q in range(4):
        if q + 1 < 4:
            acps[(q + 1) % 2] = pltpu.async_copy(
                adjv.at[s1.at[pl.ds((q + 1) * 80, 80)]],
                adjs[(q + 1) % 2], asems[(q + 1) % 2])
        acps[q % 2].wait()
        a2 = adjs[q % 2]

        def ext_body(g, _, q=q, a2=a2):
            k = lane + g * 16
            slot = lax.div(k, c25)
            j = k - slot * _NS0
            vals = plsc.load_gather(a2, [slot, j])
            s2[pl.ds(pl.multiple_of(q * 2000 + g * 16, 8), 16)] = vals
            return 0
        lax.fori_loop(0, 2000 // 16, ext_body, 0)

    for c in hcps:
        c.wait()

    rbs = (rb0, rb1, rb2, rb3)
    sts, sgs, sss = (st0, st1), (sg0, sg1, sg2, sg3), (ss0, ss1)

    def start_gather(c, b):
        for doff, n in ((0, 64), (64, 64), (128, 72)):
            off = pl.multiple_of(c * _ROWS + doff, 8)
            pltpu.async_copy(feat.at[s2.at[pl.ds(off, n)]],
                             rbs[b].at[pl.ds(doff, n)], sgs[b])

    def wait_gather(b):
        pltpu.make_async_copy(feat.at[pl.ds(0, _ROWS)], rbs[b], sgs[b]).wait()

    def wait_store(b):
        pltpu.make_async_copy(sts[b], s2_o.at[pl.ds(0, _CH)], sss[b]).wait()

    def acc_chunk(rb, st):
        def seg_body(sl, _):
            rbase = sl * _NS0

            def row5_body(r5, accs):
                row = rbase + r5 * 5
                for r in range(5):
                    accs = tuple(accs[v] + rb[row + r, pl.ds(v * 16, 16)]
                                 for v in range(8))
                return accs

            accs = lax.fori_loop(
                0, _NS0 // 5, row5_body,
                tuple(jnp.zeros((16,), jnp.float32) for _ in range(8)))
            for v in range(8):
                st[sl, pl.ds(v * 16, 16)] = accs[v]
            return 0
        lax.fori_loop(0, _CH, seg_body, 0)

    start_gather(0, 0)
    start_gather(1, 1)
    start_gather(2, 2)

    def quad_body(p, _):
        for b in range(4):
            c = p * 4 + b

            @pl.when(c + 3 < _NCHUNK)
            def _():
                start_gather(c + 3, (b + 3) % 4)

            wait_gather(b)

            @pl.when(c >= 2)
            def _():
                wait_store(b % 2)

            acc_chunk(rbs[b], sts[b % 2])
            row0 = (lax.shift_right_logical(c, 2) * _B + base_b
                    + lax.shift_left(c & 3, 3))
            pltpu.async_copy(
                sts[b % 2], s2_o.at[pl.ds(pl.multiple_of(row0, 8), _CH)],
                sss[b % 2])
        return 0

    lax.fori_loop(0, _NCHUNK // 4, quad_body, 0)
    wait_store(0)
    wait_store(1)


def _sc_stage(features, adj, batch1):
    f32, i32 = jnp.float32, jnp.int32
    fn = functools.partial(
        pl.kernel,
        out_type=[
            jax.ShapeDtypeStruct((_B, _D), f32),
            jax.ShapeDtypeStruct((_B * _NS1, _D), f32),
            jax.ShapeDtypeStruct((_B * _NS1, _D), f32),
        ],
        mesh=plsc.VectorSubcoreMesh(core_axis_name="c", subcore_axis_name="s"),
        compiler_params=pltpu.CompilerParams(needs_layout_passes=False, use_tc_tiling_on_sc=False),
        scratch_types=[
            pltpu.VMEM((_NB,), i32),
            pltpu.VMEM((_NB, _D), f32),
            pltpu.VMEM((_NB, 32), i32),
            pltpu.VMEM((_S1,), i32),
            pltpu.VMEM((80, 32), i32),
            pltpu.VMEM((80, 32), i32),
            pltpu.VMEM((_S1 * _NS0,), i32),
            pltpu.VMEM((_ROWS, _D), f32),
            pltpu.VMEM((_ROWS, _D), f32),
            pltpu.VMEM((_ROWS, _D), f32),
            pltpu.VMEM((_ROWS, _D), f32),
            pltpu.VMEM((_CH, _D), f32),
            pltpu.VMEM((_CH, _D), f32),
            pltpu.SemaphoreType.DMA,
            pltpu.SemaphoreType.DMA,
            pltpu.SemaphoreType.DMA,
            pltpu.SemaphoreType.DMA,
            pltpu.SemaphoreType.DMA,
            pltpu.SemaphoreType.DMA,
            pltpu.SemaphoreType.DMA,
        ],
    )(_sc_body)
    return fn(features, adj, batch1)


def _tc_body(h0_ref, h1_ref, s2sum_ref, ws0_ref, wn0_ref, ws1_ref, wn1_ref,
             out_ref):
    ws0 = ws0_ref[...]
    wn0 = wn0_ref[...]
    h0 = h0_ref[...]
    h1 = h1_ref[...]
    nm0 = h1[:_B]
    for j in range(1, _NS1):
        nm0 = nm0 + h1[j * _B:(j + 1) * _B]
    nm0 = nm0 * (1.0 / _NS1)
    a00 = jax.nn.relu(jnp.dot(h0, ws0, preferred_element_type=jnp.float32))
    b00 = jax.nn.relu(jnp.dot(nm0, wn0, preferred_element_type=jnp.float32))
    nm1 = s2sum_ref[...] * (1.0 / _NS0)
    a01 = jax.nn.relu(jnp.dot(h1, ws0, preferred_element_type=jnp.float32))
    b01 = jax.nn.relu(jnp.dot(nm1, wn0, preferred_element_type=jnp.float32))
    nma = a01[:_B]
    nmb = b01[:_B]
    for j in range(1, _NS1):
        nma = nma + a01[j * _B:(j + 1) * _B]
        nmb = nmb + b01[j * _B:(j + 1) * _B]
    nma = nma * (1.0 / _NS1)
    nmb = nmb * (1.0 / _NS1)
    ws1 = ws1_ref[...]
    wn1 = wn1_ref[...]
    fs = (jnp.dot(a00, ws1[:_D], preferred_element_type=jnp.float32)
          + jnp.dot(b00, ws1[_D:], preferred_element_type=jnp.float32))
    fn = (jnp.dot(nma, wn1[:_D], preferred_element_type=jnp.float32)
          + jnp.dot(nmb, wn1[_D:], preferred_element_type=jnp.float32))
    out = jnp.concatenate([fs, fn], axis=1)
    norm2 = jnp.sum(out * out, axis=1, keepdims=True)
    out_ref[...] = out * jax.lax.rsqrt(jnp.maximum(norm2, 1e-24))


def _dense_stage(h0, h1, s2sum, ws0, wn0, ws1, wn1):
    return pl.pallas_call(
        _tc_body,
        out_shape=jax.ShapeDtypeStruct((_B, 2 * _D), jnp.float32),
    )(h0, h1, s2sum, ws0, wn0, ws1, wn1)


def kernel(features, adj, batch1, W_self_0, W_neigh_0, W_self_1, W_neigh_1):
    h0, h1, s2sum = _sc_stage(features, adj, batch1)
    return _dense_stage(h0, h1, s2sum, W_self_0, W_neigh_0, W_self_1, W_neigh_1)
